# baseline (speedup 1.0000x reference)
import functools

import jax
import jax.numpy as jnp
from jax import lax
from jax.experimental import pallas as pl
from jax.experimental.pallas import tpu as pltpu
from jax.experimental.pallas import tpu_sc as plsc

_NUM_CORES = 2
_NUM_SUBCORES = 16
_NUM_WORKERS = _NUM_CORES * _NUM_SUBCORES

_GATHER_CHUNK = 64


def _sc_gather(table, sentences, pos, cs, bsz, h):
    wpb = _NUM_WORKERS // bsz
    per_w = cs // wpb
    n_chunks = per_w // _GATHER_CHUNK
    mesh = plsc.VectorSubcoreMesh(core_axis_name="c", subcore_axis_name="s")

    @functools.partial(
        pl.kernel,
        mesh=mesh,
        out_type=jax.ShapeDtypeStruct((bsz, cs, h), jnp.float32),
        scratch_types=[
            pltpu.VMEM((per_w,), jnp.int32),
            pltpu.VMEM((_GATHER_CHUNK, h), jnp.float32),
            pltpu.VMEM((_GATHER_CHUNK, h), jnp.float32),
            pltpu.SemaphoreType.DMA,
            pltpu.SemaphoreType.DMA,
        ],
    )
    def gather_kernel(table_hbm, sent_hbm, out_hbm, idx_v, buf0, buf1, sem0, sem1):
        wid = lax.axis_index("s") * _NUM_CORES + lax.axis_index("c")
        b = wid // wpb
        lane = wid % wpb
        base = lane * per_w
        pltpu.sync_copy(sent_hbm.at[b, pl.ds(pos + base, per_w)], idx_v)
        bufs = (buf0, buf1)
        sems = (sem0, sem1)
        copies = []
        for c in range(n_chunks):
            copies.append(
                pltpu.async_copy(
                    table_hbm.at[idx_v.at[pl.ds(c * _GATHER_CHUNK, _GATHER_CHUNK)]],
                    bufs[c % 2],
                    sems[c % 2],
                )
            )
            if c >= 1:
                copies[c - 1].wait()
                pltpu.sync_copy(
                    bufs[(c - 1) % 2],
                    out_hbm.at[b, pl.ds(base + (c - 1) * _GATHER_CHUNK, _GATHER_CHUNK)],
                )
        copies[-1].wait()
        pltpu.sync_copy(
            bufs[(n_chunks - 1) % 2],
            out_hbm.at[b, pl.ds(base + (n_chunks - 1) * _GATHER_CHUNK, _GATHER_CHUNK)],
        )

    return gather_kernel(table, sentences)


def _mlp_mean(g3, w1, b1, w2, b2, bsz, cs, h, bs, s_total, row_offset, out_prev):
    nb = cs // bs
    off = row_offset // bs

    def body(x_ref, w1_ref, b1_ref, w2_ref, b2_ref, *rest):
        o_ref = rest[-1]
        x = x_ref[...].reshape(bsz * bs, h).astype(jnp.bfloat16)
        hid = lax.dot_general(
            x, w1_ref[...], (((1,), (1,)), ((), ())),
            preferred_element_type=jnp.float32,
        )
        hid = jnp.maximum(hid + b1_ref[...], 0.0).astype(jnp.bfloat16)
        y = lax.dot_general(
            hid, w2_ref[...], (((1,), (1,)), ((), ())),
            preferred_element_type=jnp.float32,
        )
        y = jnp.maximum(y + b2_ref[...], 0.0)
        o_ref[...] = jnp.sum(y.reshape(bsz, bs, h), axis=0) * (1.0 / bsz)

    in_specs = [
        pl.BlockSpec((bsz, bs, h), lambda i: (0, i, 0)),
        pl.BlockSpec((h, h), lambda i: (0, 0)),
        pl.BlockSpec((1, h), lambda i: (0, 0)),
        pl.BlockSpec((h, h), lambda i: (0, 0)),
        pl.BlockSpec((1, h), lambda i: (0, 0)),
    ]
    args = [g3, w1, b1, w2, b2]
    aliases = {}
    if out_prev is not None:
        in_specs.append(pl.BlockSpec(memory_space=pl.ANY))
        args.append(out_prev)
        aliases = {5: 0}
    return pl.pallas_call(
        body,
        grid=(nb,),
        in_specs=in_specs,
        out_specs=pl.BlockSpec((bs, h), lambda i: (off + i, 0)),
        out_shape=jax.ShapeDtypeStruct((s_total, h), jnp.float32),
        input_output_aliases=aliases,
        compiler_params=pltpu.CompilerParams(
            dimension_semantics=("parallel",)),
    )(*args)


def kernel(sentences_encoded, attention_mask, embed_table, W1, b1, W2, b2):
    del attention_mask
    bsz, s = sentences_encoded.shape
    h = embed_table.shape[1]
    sent = sentences_encoded.astype(jnp.int32)
    w1 = W1.astype(jnp.bfloat16)
    w2 = W2.astype(jnp.bfloat16)
    b1r = b1.reshape(1, h)
    b2r = b2.reshape(1, h)
    chunk_sizes = (1024, 2048, 2048, 2048, 1024)
    assert sum(chunk_sizes) == s
    bs = 1024
    gathered = []
    pos = 0
    for cs in chunk_sizes:
        gathered.append(_sc_gather(embed_table, sent, pos, cs, bsz, h))
        pos += cs
    out = None
    pos = 0
    for cs, g_j in zip(chunk_sizes, gathered):
        out = _mlp_mean(g_j, w1, b1r, w2, b2r, bsz, cs, h, bs, s, pos, out)
        pos += cs
    return out


# speedup vs baseline: 1.0067x; 1.0067x over previous
import functools

import jax
import jax.numpy as jnp
from jax import lax
from jax.experimental import pallas as pl
from jax.experimental.pallas import tpu as pltpu
from jax.experimental.pallas import tpu_sc as plsc

_NUM_CORES = 2
_NUM_SUBCORES = 16
_NUM_WORKERS = _NUM_CORES * _NUM_SUBCORES

_GATHER_CHUNK = 64


def _sc_gather(table, sentences, pos, cs, bsz, h):
    wpb = _NUM_WORKERS // bsz
    per_w = cs // wpb
    n_chunks = per_w // _GATHER_CHUNK
    mesh = plsc.VectorSubcoreMesh(core_axis_name="c", subcore_axis_name="s")

    @functools.partial(
        pl.kernel,
        mesh=mesh,
        out_type=jax.ShapeDtypeStruct((bsz, cs, h), jnp.float32),
        scratch_types=[
            pltpu.VMEM((per_w,), jnp.int32),
            pltpu.VMEM((_GATHER_CHUNK, h), jnp.float32),
            pltpu.VMEM((_GATHER_CHUNK, h), jnp.float32),
            pltpu.SemaphoreType.DMA,
            pltpu.SemaphoreType.DMA,
        ],
    )
    def gather_kernel(table_hbm, sent_hbm, out_hbm, idx_v, buf0, buf1, sem0, sem1):
        wid = lax.axis_index("s") * _NUM_CORES + lax.axis_index("c")
        b = wid // wpb
        lane = wid % wpb
        base = lane * per_w
        pltpu.sync_copy(sent_hbm.at[b, pl.ds(pos + base, per_w)], idx_v)
        bufs = (buf0, buf1)
        sems = (sem0, sem1)
        copies = []
        for c in range(n_chunks):
            copies.append(
                pltpu.async_copy(
                    table_hbm.at[idx_v.at[pl.ds(c * _GATHER_CHUNK, _GATHER_CHUNK)]],
                    bufs[c % 2],
                    sems[c % 2],
                )
            )
            if c >= 1:
                copies[c - 1].wait()
                pltpu.sync_copy(
                    bufs[(c - 1) % 2],
                    out_hbm.at[b, pl.ds(base + (c - 1) * _GATHER_CHUNK, _GATHER_CHUNK)],
                )
        copies[-1].wait()
        pltpu.sync_copy(
            bufs[(n_chunks - 1) % 2],
            out_hbm.at[b, pl.ds(base + (n_chunks - 1) * _GATHER_CHUNK, _GATHER_CHUNK)],
        )

    return gather_kernel(table, sentences)


def _mlp_mean(g3, w1, b1, w2, b2, bsz, cs, h, bs, s_total, row_offset, out_prev):
    nb = cs // bs
    off = row_offset // bs

    def body(x_ref, w1_ref, b1_ref, w2_ref, b2_ref, *rest):
        o_ref = rest[-1]
        x = x_ref[...].reshape(bsz * bs, h).astype(jnp.bfloat16)
        hid = lax.dot_general(
            x, w1_ref[...], (((1,), (1,)), ((), ())),
            preferred_element_type=jnp.float32,
        )
        hid = jnp.maximum(hid + b1_ref[...], 0.0).astype(jnp.bfloat16)
        y = lax.dot_general(
            hid, w2_ref[...], (((1,), (1,)), ((), ())),
            preferred_element_type=jnp.float32,
        )
        y = jnp.maximum(y + b2_ref[...], 0.0)
        o_ref[...] = jnp.sum(y.reshape(bsz, bs, h), axis=0) * (1.0 / bsz)

    in_specs = [
        pl.BlockSpec((bsz, bs, h), lambda i: (0, i, 0)),
        pl.BlockSpec((h, h), lambda i: (0, 0)),
        pl.BlockSpec((1, h), lambda i: (0, 0)),
        pl.BlockSpec((h, h), lambda i: (0, 0)),
        pl.BlockSpec((1, h), lambda i: (0, 0)),
    ]
    args = [g3, w1, b1, w2, b2]
    aliases = {}
    if out_prev is not None:
        in_specs.append(pl.BlockSpec(memory_space=pl.ANY))
        args.append(out_prev)
        aliases = {5: 0}
    return pl.pallas_call(
        body,
        grid=(nb,),
        in_specs=in_specs,
        out_specs=pl.BlockSpec((bs, h), lambda i: (off + i, 0)),
        out_shape=jax.ShapeDtypeStruct((s_total, h), jnp.float32),
        input_output_aliases=aliases,
        compiler_params=pltpu.CompilerParams(
            dimension_semantics=("parallel",)),
    )(*args)


def kernel(sentences_encoded, attention_mask, embed_table, W1, b1, W2, b2):
    del attention_mask
    bsz, s = sentences_encoded.shape
    h = embed_table.shape[1]
    sent = sentences_encoded.astype(jnp.int32)
    w1 = W1.astype(jnp.bfloat16)
    w2 = W2.astype(jnp.bfloat16)
    b1r = b1.reshape(1, h)
    b2r = b2.reshape(1, h)
    chunk_sizes = (1024, 1024, 2048, 2048, 2048)
    assert sum(chunk_sizes) == s
    bs = 1024
    gathered = []
    pos = 0
    for cs in chunk_sizes:
        gathered.append(_sc_gather(embed_table, sent, pos, cs, bsz, h))
        pos += cs
    out = None
    pos = 0
    for cs, g_j in zip(chunk_sizes, gathered):
        out = _mlp_mean(g_j, w1, b1r, w2, b2r, bsz, cs, h, bs, s, pos, out)
        pos += cs
    return out
